# Initial kernel scaffold; baseline (speedup 1.0000x reference)
import jax
import jax.numpy as jnp
from jax.experimental import pallas as pl


def kernel(features, cluster_features, xws, yws, edge_index, raw_affine, Wg0, as0, ad0, Wo0, bo0, Wg1, as1, ad1, Wo1, bo1, Wg2, as2, ad2, Wo2, bo2, Wg3, as3, ad3, Wo3, bo3, Wsrc, bsrc, Wdst, bdst, p1, cW1, cb1, p2, cW2, cb2):
    raise NotImplementedError("write your pallas kernel here")


# speedup vs baseline: 13.1724x; 13.1724x over previous
import functools

import jax
import jax.numpy as jnp
from jax import lax
from jax.experimental import pallas as pl
from jax.experimental.pallas import tpu as pltpu
from jax.experimental.pallas import tpu_sc as plsc

N = 10000
E = 320000
N2 = 10240
NTILE = 16
NSC = 2
K = 128
NCH = 2560
E_PAD = NCH * K
CPT16 = NCH // NTILE
CPT32 = NCH // 32
RB = N2 // NTILE
NRED = N2 // NTILE

_MESH = plsc.VectorSubcoreMesh(core_axis_name="c", subcore_axis_name="s")
_SC_PARAMS = pltpu.CompilerParams(needs_layout_passes=False)


def _head_body(di, do, split, x_ref, wg_ref, as_ref, ad_ref, wob_ref,
               es_ref, ed_ref, *rest):
    i = pl.program_id(0)
    d2 = do // 2
    h = jnp.dot(x_ref[...], wg_ref[...], preferred_element_type=jnp.float32)
    es = jnp.sum(h * as_ref[0][None, :], axis=1)
    ed = jnp.sum(h * ad_ref[0][None, :], axis=1)
    es_ref[...] = es[:, None]
    ed_ref[...] = ed[:, None]
    g_ref, gm_ref = rest
    if split:
        g_ref[0] = jnp.dot(h, wob_ref[...][:, :d2],
                           preferred_element_type=jnp.float32)
        g_ref[1] = jnp.dot(h, wob_ref[...][:, d2:],
                           preferred_element_type=jnp.float32)
    else:
        g_ref[...] = jnp.dot(h, wob_ref[...],
                             preferred_element_type=jnp.float32)

    @pl.when(i == 0)
    def _():
        gm_ref[0, 0] = -3.0e38
    gm_ref[0, 0] = jnp.maximum(gm_ref[0, 0], jnp.max(es))


def _tc_head(x, wg, a_s, a_d, wob, di, do, split):
    B = 2000
    gw = do // 2 if split else do
    gspec = (pl.BlockSpec((2, B, gw), lambda i: (0, i, 0)) if split
             else pl.BlockSpec((B, gw), lambda i: (i, 0)))
    gshape = ((2, N, gw) if split else (N, gw))
    grid = (N // B,)
    return pl.pallas_call(
        functools.partial(_head_body, di, do, split),
        grid=grid,
        in_specs=[
            pl.BlockSpec((B, di), lambda i: (i, 0)),
            pl.BlockSpec((di, di), lambda i: (0, 0)),
            pl.BlockSpec((1, di), lambda i: (0, 0)),
            pl.BlockSpec((1, di), lambda i: (0, 0)),
            pl.BlockSpec((di, do), lambda i: (0, 0)),
        ],
        out_specs=[
            pl.BlockSpec((B, 1), lambda i: (i, 0)),
            pl.BlockSpec((B, 1), lambda i: (i, 0)),
            gspec,
            pl.BlockSpec(memory_space=pltpu.SMEM),
        ],
        out_shape=[
            jax.ShapeDtypeStruct((N, 1), jnp.float32),
            jax.ShapeDtypeStruct((N, 1), jnp.float32),
            jax.ShapeDtypeStruct(gshape, jnp.float32),
            jax.ShapeDtypeStruct((1, 1), jnp.float32),
        ],
    )(x, wg, a_s, a_d, wob)


def _densum_body(d16_ref, o_ref):
    o_ref[...] = jnp.sum(d16_ref[:, 0, :], axis=0)[:, None]


def _tc_densum(den16):
    return pl.pallas_call(
        _densum_body,
        out_shape=jax.ShapeDtypeStruct((N2, 1), jnp.float32),
    )(den16)


def _out_body(concat_acc, x_ref, a0_ref, a1_ref, den_ref, wot_ref, bo_ref,
              o_ref):
    den = jnp.maximum(den_ref[...][:, 0], 1e-30)
    if concat_acc:
        acc = jnp.concatenate([a0_ref[0], a1_ref[0]], axis=1)
    else:
        acc = a0_ref[0] + a1_ref[0]
    agg = acc / den[:, None]
    y = jnp.dot(x_ref[...], wot_ref[...], preferred_element_type=jnp.float32)
    o_ref[...] = jnp.maximum(y + agg + bo_ref[0][None, :], 0.0)


def _tc_out(x, acc, den, wot, bo, di, do, concat_acc):
    B = 2000
    d2 = acc.shape[2]
    return pl.pallas_call(
        functools.partial(_out_body, concat_acc),
        grid=(N // B,),
        in_specs=[
            pl.BlockSpec((B, di), lambda i: (i, 0)),
            pl.BlockSpec((1, B, d2), lambda i: (0, i, 0)),
            pl.BlockSpec((1, B, d2), lambda i: (1, i, 0)),
            pl.BlockSpec((B, 1), lambda i: (i, 0)),
            pl.BlockSpec((di, do), lambda i: (0, 0)),
            pl.BlockSpec((1, do), lambda i: (0, 0)),
        ],
        out_specs=pl.BlockSpec((B, do), lambda i: (i, 0)),
        out_shape=jax.ShapeDtypeStruct((N, do), jnp.float32),
    )(x, acc, acc, den, wot, bo)


def _tab_body(x_ref, ws_ref, bs_ref, wd_ref, bd_ref, xw_ref, yw_ref,
              p1_ref, w1_ref, b1_ref, t_ref):
    p1 = p1_ref[0]
    w1 = w1_ref[...]
    xw = xw_ref[...]
    yw = yw_ref[...]
    sf = jnp.dot(x_ref[...], ws_ref[...], preferred_element_type=jnp.float32)
    ts = jnp.concatenate([sf + bs_ref[0][None, :], xw, yw], axis=1)
    us = jnp.where(ts >= 0, ts, ts * p1[:8][None, :])
    hs = jnp.dot(us, w1[:8, :], preferred_element_type=jnp.float32) \
        + b1_ref[0][None, :]
    df = jnp.dot(x_ref[...], wd_ref[...], preferred_element_type=jnp.float32)
    td = jnp.concatenate([df + bd_ref[0][None, :], xw, yw], axis=1)
    ud = jnp.where(td >= 0, td, td * p1[8:][None, :])
    hd = jnp.dot(ud, w1[8:, :], preferred_element_type=jnp.float32)
    t_ref[0] = hs
    t_ref[1] = hd


def _tc_tables(x, ws, bs, wd, bd, xw, yw, p1, w1, b1):
    B = 2000
    di = x.shape[1]
    return pl.pallas_call(
        _tab_body,
        grid=(N // B,),
        in_specs=[
            pl.BlockSpec((B, di), lambda i: (i, 0)),
            pl.BlockSpec((di, 6), lambda i: (0, 0)),
            pl.BlockSpec((1, 6), lambda i: (0, 0)),
            pl.BlockSpec((di, 6), lambda i: (0, 0)),
            pl.BlockSpec((1, 6), lambda i: (0, 0)),
            pl.BlockSpec((B, 1), lambda i: (i, 0)),
            pl.BlockSpec((B, 1), lambda i: (i, 0)),
            pl.BlockSpec((1, 16), lambda i: (0, 0)),
            pl.BlockSpec((16, 8), lambda i: (0, 0)),
            pl.BlockSpec((1, 8), lambda i: (0, 0)),
        ],
        out_specs=pl.BlockSpec((2, B, 8), lambda i: (0, i, 0)),
        out_shape=jax.ShapeDtypeStruct((2, N, 8), jnp.float32),
    )(x, ws, bs, wd, bd, xw, yw, p1, w1, b1)


def _fin_body(sp_ref, cp_ref, o_ref):
    sv = jnp.sum(sp_ref[:, 0, :], axis=0)
    cv = jnp.sum(cp_ref[:, 0, :], axis=0)
    pd = sv / jnp.maximum(cv, 1.0)
    o_ref[...] = pd[None, :N]


def _tc_finalize(sp, cp):
    return pl.pallas_call(
        _fin_body,
        out_shape=jax.ShapeDtypeStruct((1, N), jnp.float32),
    )(sp, cp)


_Z16F = functools.partial(jnp.zeros, (16,), jnp.float32)


def _sc_layer_body(d2, esplit,
                   es_h, ed_h, gm_h, src2_h, dst2_h, g_h,
                   den_h, acc_h,
                   es_v, ed_v, gm_v, src_c, dst_c, w_c, den_v, row_v,
                   acc_sh):
    c = lax.axis_index("c")
    s = lax.axis_index("s")
    iota16 = lax.iota(jnp.int32, 16)
    zf = _Z16F()
    zi = jnp.zeros((16,), jnp.int32)

    pltpu.sync_copy(es_h, es_v)
    pltpu.sync_copy(ed_h, ed_v)
    pltpu.sync_copy(gm_h, gm_v)

    def zden(i, _):
        den_v[0, pl.ds(i * 16, 16)] = zf
        return 0
    lax.fori_loop(0, N2 // 16, zden, 0)

    def zrow(k, _):
        for u in range(d2 // 16):
            row_v[k, pl.ds(u * 16, 16)] = zf
        return 0
    lax.fori_loop(0, K, zrow, 0)

    for r0 in range(0, RB, K):
        pltpu.sync_copy(row_v, acc_sh.at[pl.ds(s * RB + r0, K)])
    plsc.subcore_barrier()

    gmv = gm_v[...]

    def chunk(j, _):
        if esplit:
            row = c * (NCH // 2) + s * CPT32 + j
        else:
            row = s * CPT16 + j
        pltpu.sync_copy(src2_h.at[row], src_c)
        pltpu.sync_copy(dst2_h.at[row], dst_c)

        base = row * K
        for t in range(K // 16):
            sidx = src_c[0, pl.ds(t * 16, 16)]
            didx = dst_c[0, pl.ds(t * 16, 16)]
            esg = plsc.load_gather(es_v, [sidx])
            edg = plsc.load_gather(ed_v, [didx])
            z = esg + edg
            e = jnp.maximum(z, 0.2 * z)
            m = jnp.maximum(edg + gmv, 0.0)
            w = jnp.exp(e - m)
            gid = base + t * 16 + iota16
            w = jnp.where(gid < E, w, 0.0)
            w_c[0, pl.ds(t * 16, 16)] = w
            plsc.addupdate_scatter(den_v, [zi, didx], w)
            if not esplit:
                src_c[0, pl.ds(t * 16, 16)] = sidx + c * N
        pltpu.sync_copy(g_h.at[src_c.at[0]], row_v)

        def scale(k, _):
            wb = plsc.load_gather(
                w_c, [zi, jnp.full((16,), k, jnp.int32)])
            for u in range(d2 // 16):
                row_v[k, pl.ds(u * 16, 16)] = row_v[k, pl.ds(u * 16, 16)] * wb
            return 0
        lax.fori_loop(0, K, scale, 0)
        pltpu.sync_copy(row_v, acc_sh.at[dst_c.at[0]], add=True)
        return 0
    lax.fori_loop(0, CPT32 if esplit else CPT16, chunk, 0)

    if esplit:
        pltpu.sync_copy(den_v, den_h.at[s * NSC + c])
    else:
        @pl.when(c == 0)
        def _():
            pltpu.sync_copy(den_v, den_h.at[s])

    plsc.subcore_barrier()

    pltpu.sync_copy(acc_sh.at[pl.ds(s * RB, RB)],
                    acc_h.at[c, pl.ds(s * RB, RB)])


def _sc_layer(es, ed, gm16, src2, dst2, g, d2, esplit):
    nden = 2 * NTILE if esplit else NTILE
    fn = pl.kernel(
        functools.partial(_sc_layer_body, d2, esplit),
        out_type=(
            jax.ShapeDtypeStruct((nden, 1, N2), jnp.float32),
            jax.ShapeDtypeStruct((2, N2, d2), jnp.float32),
        ),
        mesh=_MESH,
        compiler_params=_SC_PARAMS,
        scratch_types=[
            pltpu.VMEM((N,), jnp.float32),
            pltpu.VMEM((N,), jnp.float32),
            pltpu.VMEM((16,), jnp.float32),
            pltpu.VMEM((1, K), jnp.int32),
            pltpu.VMEM((1, K), jnp.int32),
            pltpu.VMEM((1, K), jnp.float32),
            pltpu.VMEM((1, N2), jnp.float32),
            pltpu.VMEM((K, d2), jnp.float32),
            pltpu.VMEM_SHARED((N2, d2), jnp.float32),
        ],
    )
    return fn(es, ed, gm16, src2, dst2, g)


def _sc_fsrc_body(tabs_h, src2_h, u_h, tab_v, ic, u_v):
    c = lax.axis_index("c")
    s = lax.axis_index("s")
    wid = s * NSC + c
    pltpu.sync_copy(tabs_h, tab_v)

    def chunk(j, _):
        row = wid * CPT32 + j
        pltpu.sync_copy(src2_h.at[row], ic)
        for t in range(K // 16):
            sidx8 = ic[0, pl.ds(t * 16, 16)] * 8
            for f in range(8):
                u_v[f, pl.ds(t * 16, 16)] = plsc.load_gather(
                    tab_v, [sidx8 + f])
        pltpu.sync_copy(u_v, u_h.at[row])
        return 0
    lax.fori_loop(0, CPT32, chunk, 0)


def _sc_fsrc(tabs, src2):
    fn = pl.kernel(
        _sc_fsrc_body,
        out_type=jax.ShapeDtypeStruct((NCH, 8, K), jnp.float32),
        mesh=_MESH,
        compiler_params=_SC_PARAMS,
        scratch_types=[
            pltpu.VMEM((N * 8,), jnp.float32),
            pltpu.VMEM((1, K), jnp.int32),
            pltpu.VMEM((8, K), jnp.float32),
        ],
    )
    return fn(tabs, src2)


def _sc_fdst_body(tabd_h, u2_h, dst2_h, ra2_h, wt_h,
                  p0_h, p1_h, sp_h, cp_h,
                  tab_v, ic, ra_v, u_v, wt_v, p0_v, p1_v, s_v, c_v):
    c = lax.axis_index("c")
    s = lax.axis_index("s")
    wid = s * NSC + c
    iota16 = lax.iota(jnp.int32, 16)
    zf = _Z16F()
    zi = jnp.zeros((16,), jnp.int32)

    pltpu.sync_copy(tabd_h, tab_v)
    pltpu.sync_copy(wt_h, wt_v)

    def zv(i, _):
        s_v[0, pl.ds(i * 16, 16)] = zf
        c_v[0, pl.ds(i * 16, 16)] = zf
        return 0
    lax.fori_loop(0, N2 // 16, zv, 0)

    def chunk(j, _):
        row = wid * CPT32 + j
        pltpu.sync_copy(dst2_h.at[row], ic)
        pltpu.sync_copy(ra2_h.at[row], ra_v)
        pltpu.sync_copy(u2_h.at[row], u_v)
        p2b = [wt_v[f, pl.ds(0, 16)] for f in range(8)]
        w2b0 = [wt_v[8 + 2 * f, pl.ds(0, 16)] for f in range(8)]
        w2b1 = [wt_v[9 + 2 * f, pl.ds(0, 16)] for f in range(8)]
        cb20 = wt_v[24, pl.ds(0, 16)]
        cb21 = wt_v[25, pl.ds(0, 16)]
        base = row * K
        for t in range(K // 16):
            didx = ic[0, pl.ds(t * 16, 16)]
            didx8 = didx * 8
            h20 = cb20
            h21 = cb21
            for f in range(8):
                af = u_v[f, pl.ds(t * 16, 16)]
                bf = plsc.load_gather(tab_v, [didx8 + f])
                h1 = af + bf
                v = jnp.maximum(h1, 0.0) + p2b[f] * jnp.minimum(h1, 0.0)
                h20 = h20 + v * w2b0[f]
                h21 = h21 + v * w2b1[f]
            mx = jnp.maximum(h20, h21)
            e0 = jnp.exp(h20 - mx)
            e1 = jnp.exp(h21 - mx)
            tot = e0 + e1
            pr0 = e0 / tot
            pr1 = e1 / tot
            p0_v[pl.ds(t * 16, 16)] = h20
            p1_v[pl.ds(t * 16, 16)] = h21
            rab = ra_v[0, pl.ds(t * 16, 16)]
            gid = base + t * 16 + iota16
            mask = gid < E
            msg = jnp.where(mask, rab * (pr1 - pr0), 0.0)
            one = jnp.where(mask, 1.0, 0.0)
            plsc.addupdate_scatter(s_v, [zi, didx], msg)
            plsc.addupdate_scatter(c_v, [zi, didx], one)
        pltpu.sync_copy(p0_v, p0_h.at[pl.ds(row * K, K)])
        pltpu.sync_copy(p1_v, p1_h.at[pl.ds(row * K, K)])
        return 0
    lax.fori_loop(0, CPT32, chunk, 0)

    pltpu.sync_copy(s_v, sp_h.at[wid])
    pltpu.sync_copy(c_v, cp_h.at[wid])


def _sc_fdst(tabd, u2, dst2, ra2, wt):
    fn = pl.kernel(
        _sc_fdst_body,
        out_type=(
            jax.ShapeDtypeStruct((E_PAD,), jnp.float32),
            jax.ShapeDtypeStruct((E_PAD,), jnp.float32),
            jax.ShapeDtypeStruct((2 * NTILE, 1, N2), jnp.float32),
            jax.ShapeDtypeStruct((2 * NTILE, 1, N2), jnp.float32),
        ),
        mesh=_MESH,
        compiler_params=_SC_PARAMS,
        scratch_types=[
            pltpu.VMEM((N * 8,), jnp.float32),
            pltpu.VMEM((1, K), jnp.int32),
            pltpu.VMEM((1, K), jnp.float32),
            pltpu.VMEM((8, K), jnp.float32),
            pltpu.VMEM((32, 16), jnp.float32),
            pltpu.VMEM((K,), jnp.float32),
            pltpu.VMEM((K,), jnp.float32),
            pltpu.VMEM((1, N2), jnp.float32),
            pltpu.VMEM((1, N2), jnp.float32),
        ],
    )
    return fn(tabd, u2, dst2, ra2, wt)


def kernel(features, cluster_features, xws, yws, edge_index, raw_affine,
           Wg0, as0, ad0, Wo0, bo0, Wg1, as1, ad1, Wo1, bo1,
           Wg2, as2, ad2, Wo2, bo2, Wg3, as3, ad3, Wo3, bo3,
           Wsrc, bsrc, Wdst, bdst, p1, cW1, cb1, p2, cW2, cb2):
    src = edge_index[0]
    dst = edge_index[1]
    padi = jnp.zeros((E_PAD - E,), jnp.int32)
    src2 = jnp.concatenate([src, padi]).reshape(NCH, 1, K)
    dst2 = jnp.concatenate([dst, padi]).reshape(NCH, 1, K)
    ra2 = jnp.concatenate(
        [raw_affine, jnp.zeros((E_PAD - E,), jnp.float32)]).reshape(NCH, 1, K)

    x = jnp.concatenate([features, cluster_features], axis=1)
    layers = [(Wg0, as0, ad0, Wo0, bo0), (Wg1, as1, ad1, Wo1, bo1),
              (Wg2, as2, ad2, Wo2, bo2), (Wg3, as3, ad3, Wo3, bo3)]
    for Wg, a_s, a_d, Wo, bo in layers:
        di = Wg.shape[0]
        do = Wo.shape[1]
        split = do == 256
        es, ed, g, gm = _tc_head(x, Wg, a_s.reshape(1, di),
                                 a_d.reshape(1, di), Wo[di:], di, do, split)
        if split:
            g = g.reshape(2 * N, do // 2)
        gm16 = jnp.broadcast_to(gm[0, 0], (16,))
        den16, acc = _sc_layer(es.reshape(N), ed.reshape(N), gm16,
                               src2, dst2, g,
                               do // 2 if split else do, not split)
        x = _tc_out(x, acc, _tc_densum(den16), Wo[:di],
                    bo.reshape(1, do), di, do, split)

    tab = _tc_tables(x, Wsrc, bsrc.reshape(1, 6), Wdst, bdst.reshape(1, 6),
                     xws.reshape(N, 1), yws.reshape(N, 1),
                     p1.reshape(1, 16),
                     cW1, cb1.reshape(1, 8))
    wt = jnp.concatenate([p2, cW2.reshape(-1), cb2,
                          jnp.zeros((6,), jnp.float32)])
    wt2 = jnp.broadcast_to(wt[:, None], (32, 16))
    u2 = _sc_fsrc(tab[0].reshape(N * 8), src2)
    p0, p1o, sp, cp = _sc_fdst(tab[1].reshape(N * 8), u2, dst2, ra2, wt2)
    pred_conn = jnp.stack([p0[:E], p1o[:E]], axis=1)
    pred_den = _tc_finalize(sp, cp)
    return pred_conn, pred_den.reshape(N)
